# Initial kernel scaffold; baseline (speedup 1.0000x reference)
import jax
import jax.numpy as jnp
from jax.experimental import pallas as pl


def kernel(x, emb_table, fc_w, fc_b):
    raise NotImplementedError("write your pallas kernel here")


# speedup vs baseline: 5.3776x; 5.3776x over previous
import functools

import jax
import jax.numpy as jnp
from jax import lax
from jax.experimental import pallas as pl
from jax.experimental.pallas import tpu as pltpu
from jax.experimental.pallas import tpu_sc as plsc

VOCAB = 100000
EMB = 300
NUM_CLASSES = 20
BATCH = 4096
TIME = 200

C_PAD = 32
NC, NS = 2, 16
NW = NC * NS
B_PER_W = BATCH // NW
T_HALF = TIME // 2


_ROWS_BLK = 2000


def _proj_body(tab_ref, w_ref, b_ref, out_ref):
    out_ref[...] = lax.dot_general(
        tab_ref[...], w_ref[...],
        (((1,), (1,)), ((), ())),
        preferred_element_type=jnp.float32,
    ) + b_ref[...]


def _project_table(emb_table, fc_w_pad, fc_b_pad):
    return pl.pallas_call(
        _proj_body,
        grid=(VOCAB // _ROWS_BLK,),
        in_specs=[
            pl.BlockSpec((_ROWS_BLK, EMB), lambda i: (i, 0)),
            pl.BlockSpec((C_PAD, EMB), lambda i: (0, 0)),
            pl.BlockSpec((1, C_PAD), lambda i: (0, 0)),
        ],
        out_specs=pl.BlockSpec((_ROWS_BLK, C_PAD), lambda i: (i, 0)),
        out_shape=jax.ShapeDtypeStruct((VOCAB, C_PAD), jnp.float32),
    )(emb_table, fc_w_pad, fc_b_pad)


def _sc_body(x_hbm, proj_hbm, out_hbm, idx_v, rows_v, out_v, sem):
    wid = lax.axis_index("s") * NC + lax.axis_index("c")
    base = wid * B_PER_W

    pltpu.sync_copy(x_hbm.at[pl.ds(base, B_PER_W)], idx_v)

    def do_row(b, _):
        cp0 = pltpu.async_copy(proj_hbm.at[idx_v.at[b, 0]], rows_v.at[0], sem)
        cp1 = pltpu.async_copy(proj_hbm.at[idx_v.at[b, 1]], rows_v.at[1], sem)
        cp0.wait()
        cp1.wait()

        def accum(t, carry):
            a0, a1, a2, a3 = carry
            a0 = a0 + rows_v[0, t, 0:16]
            a1 = a1 + rows_v[0, t, 16:32]
            a2 = a2 + rows_v[1, t, 0:16]
            a3 = a3 + rows_v[1, t, 16:32]
            return a0, a1, a2, a3

        z = jnp.zeros((16,), jnp.float32)
        a0, a1, a2, a3 = lax.fori_loop(0, T_HALF, accum, (z, z, z, z),
                                       unroll=4)
        out_v[b, 0:16] = a0 + a2
        out_v[b, 16:32] = a1 + a3
        return 0

    lax.fori_loop(0, B_PER_W, do_row, 0)

    pltpu.sync_copy(out_v, out_hbm.at[pl.ds(base, B_PER_W)])


@functools.cache
def _make_sc_kernel():
    return pl.kernel(
        _sc_body,
        out_type=jax.ShapeDtypeStruct((BATCH, C_PAD), jnp.float32),
        mesh=plsc.VectorSubcoreMesh(core_axis_name="c", subcore_axis_name="s"),
        scratch_types=[
            pltpu.VMEM((B_PER_W, 2, T_HALF), jnp.int32),
            pltpu.VMEM((2, T_HALF, C_PAD), jnp.float32),
            pltpu.VMEM((B_PER_W, C_PAD), jnp.float32),
            pltpu.SemaphoreType.DMA,
        ],
        compiler_params=pltpu.CompilerParams(use_tc_tiling_on_sc=False),
    )


def kernel(x, emb_table, fc_w, fc_b):
    fc_w_pad = jnp.zeros((C_PAD, EMB), jnp.float32).at[:NUM_CLASSES].set(fc_w)
    fc_w_pad = fc_w_pad * (1.0 / TIME)
    fc_b_pad = jnp.zeros((1, C_PAD), jnp.float32).at[0, :NUM_CLASSES].set(
        fc_b * (1.0 / TIME))
    proj = _project_table(emb_table, fc_w_pad, fc_b_pad)
    x3 = x.reshape(BATCH, 2, T_HALF)
    out = _make_sc_kernel()(x3, proj)
    return out[:, :NUM_CLASSES]


# speedup vs baseline: 6.3714x; 1.1848x over previous
import functools

import jax
import jax.numpy as jnp
from jax import lax
from jax.experimental import pallas as pl
from jax.experimental.pallas import tpu as pltpu
from jax.experimental.pallas import tpu_sc as plsc

VOCAB = 100000
EMB = 300
NUM_CLASSES = 20
BATCH = 4096
TIME = 200

C_PAD = 32
NC, NS = 2, 16
NW = NC * NS
B_PER_W = BATCH // NW
T_HALF = TIME // 2


_ROWS_BLK = 2000


def _proj_body(tab_ref, w_ref, b_ref, out_ref):
    out_ref[...] = lax.dot_general(
        tab_ref[...], w_ref[...],
        (((1,), (1,)), ((), ())),
        preferred_element_type=jnp.float32,
    ) + b_ref[...]


def _project_table(emb_table, fc_w_pad, fc_b_pad):
    return pl.pallas_call(
        _proj_body,
        grid=(VOCAB // _ROWS_BLK,),
        in_specs=[
            pl.BlockSpec((_ROWS_BLK, EMB), lambda i: (i, 0)),
            pl.BlockSpec((C_PAD, EMB), lambda i: (0, 0)),
            pl.BlockSpec((1, C_PAD), lambda i: (0, 0)),
        ],
        out_specs=pl.BlockSpec((_ROWS_BLK, C_PAD), lambda i: (i, 0)),
        out_shape=jax.ShapeDtypeStruct((VOCAB, C_PAD), jnp.float32),
    )(emb_table, fc_w_pad, fc_b_pad)


def _sc_body(x_hbm, proj_hbm, out_hbm, idx_v, rows_v, out_v, sem0, sem1):
    wid = lax.axis_index("s") * NC + lax.axis_index("c")
    base = wid * B_PER_W
    sems = (sem0, sem1)

    pltpu.sync_copy(x_hbm.at[pl.ds(base, B_PER_W)], idx_v)

    def fire(b, buf):
        pltpu.async_copy(proj_hbm.at[idx_v.at[b, 0]], rows_v.at[buf, 0],
                         sems[buf])
        pltpu.async_copy(proj_hbm.at[idx_v.at[b, 1]], rows_v.at[buf, 1],
                         sems[buf])

    def drain(b, buf):
        pltpu.make_async_copy(proj_hbm.at[idx_v.at[b, 0]], rows_v.at[buf, 0],
                              sems[buf]).wait()
        pltpu.make_async_copy(proj_hbm.at[idx_v.at[b, 1]], rows_v.at[buf, 1],
                              sems[buf]).wait()

    def reduce(b, buf):
        def accum(t, carry):
            a0, a1, a2, a3 = carry
            a0 = a0 + rows_v[buf, 0, t, 0:16]
            a1 = a1 + rows_v[buf, 0, t, 16:32]
            a2 = a2 + rows_v[buf, 1, t, 0:16]
            a3 = a3 + rows_v[buf, 1, t, 16:32]
            return a0, a1, a2, a3

        z = jnp.zeros((16,), jnp.float32)
        a0, a1, a2, a3 = lax.fori_loop(0, T_HALF, accum, (z, z, z, z),
                                       unroll=4)
        out_v[b, 0:16] = a0 + a2
        out_v[b, 16:32] = a1 + a3

    fire(0, 0)

    def do_pair(g, _):
        b0 = 2 * g
        fire(b0 + 1, 1)
        drain(b0, 0)
        reduce(b0, 0)

        @pl.when(g < B_PER_W // 2 - 1)
        def _():
            fire(b0 + 2, 0)

        drain(b0 + 1, 1)
        reduce(b0 + 1, 1)
        return 0

    lax.fori_loop(0, B_PER_W // 2, do_pair, 0)

    pltpu.sync_copy(out_v, out_hbm.at[pl.ds(base, B_PER_W)])


@functools.cache
def _make_sc_kernel():
    return pl.kernel(
        _sc_body,
        out_type=jax.ShapeDtypeStruct((BATCH, C_PAD), jnp.float32),
        mesh=plsc.VectorSubcoreMesh(core_axis_name="c", subcore_axis_name="s"),
        scratch_types=[
            pltpu.VMEM((B_PER_W, 2, T_HALF), jnp.int32),
            pltpu.VMEM((2, 2, T_HALF, C_PAD), jnp.float32),
            pltpu.VMEM((B_PER_W, C_PAD), jnp.float32),
            pltpu.SemaphoreType.DMA,
            pltpu.SemaphoreType.DMA,
        ],
        compiler_params=pltpu.CompilerParams(use_tc_tiling_on_sc=False),
    )


def kernel(x, emb_table, fc_w, fc_b):
    fc_w_pad = jnp.zeros((C_PAD, EMB), jnp.float32).at[:NUM_CLASSES].set(fc_w)
    fc_w_pad = fc_w_pad * (1.0 / TIME)
    fc_b_pad = jnp.zeros((1, C_PAD), jnp.float32).at[0, :NUM_CLASSES].set(
        fc_b * (1.0 / TIME))
    proj = _project_table(emb_table, fc_w_pad, fc_b_pad)
    x3 = x.reshape(BATCH, 2, T_HALF)
    out = _make_sc_kernel()(x3, proj)
    return out[:, :NUM_CLASSES]
